# baseline (speedup 1.0000x reference)
import math
import functools

import jax
import jax.numpy as jnp
from jax import lax
from jax.experimental import pallas as pl
from jax.experimental.pallas import tpu as pltpu
from jax.experimental.pallas import tpu_sc as plsc

N_NODES_C = 10000
N_EDGES_C = 160000
N_GRAPHS_C = 16
TIME_DIM_C = 16

EDGE_BLOCK = 3200

_NC, _NS = 2, 16
_NW = _NC * _NS
_SUB = 128
_NBLK = N_EDGES_C // _SUB
_NPAD = 10240
_RPT = _NPAD // _NS
_ZROWS = 128
_ROWW = 80

_sc_mesh = plsc.VectorSubcoreMesh(core_axis_name="c", subcore_axis_name="s")


def _silu(x):
    return x * jax.nn.sigmoid(x)


def _timestep_embedding(timesteps, dim, max_period=10000):
    half = dim // 2
    freqs = jnp.exp(-math.log(max_period) * jnp.arange(0, half, dtype=jnp.float32) / half)
    args = timesteps[:, None].astype(jnp.float32) * freqs[None]
    return jnp.concatenate([jnp.cos(args), jnp.sin(args)], axis=-1)


@functools.partial(
    pl.kernel,
    out_type=jax.ShapeDtypeStruct((_NC, _NPAD, _ROWW), jnp.float32),
    mesh=_sc_mesh,
    scratch_types=[
        pltpu.VMEM((1, _SUB), jnp.int32),
        pltpu.VMEM((_SUB, _ROWW), jnp.float32),
        pltpu.VMEM((_ZROWS, _ROWW), jnp.float32),
        pltpu.VMEM_SHARED((_NPAD, _ROWW), jnp.float32),
    ],
    compiler_params=pltpu.CompilerParams(use_tc_tiling_on_sc=False),
)
def _sc_scatter(medge, row2, out, idx_v, dbuf, zbuf, acc):
    cid = lax.axis_index("c")
    sid = lax.axis_index("s")
    wid = sid * _NC + cid

    def zrow(r, carry):
        for q in range(_ROWW // 16):
            zbuf[r, pl.ds(q * 16, 16)] = jnp.zeros((16,), jnp.float32)
        return carry
    lax.fori_loop(0, _ZROWS, zrow, 0)
    for q in range(_RPT // _ZROWS):
        pltpu.sync_copy(zbuf, acc.at[pl.ds(sid * _RPT + q * _ZROWS, _ZROWS)])
    plsc.subcore_barrier()

    startb = wid * (_NBLK // _NW) + jnp.minimum(wid, _NBLK % _NW)
    endb = startb + (_NBLK // _NW) + jnp.where(wid < _NBLK % _NW, 1, 0)

    def blk_body(bi, carry):
        pltpu.sync_copy(row2.at[pl.ds(bi, 1)], idx_v)
        pltpu.sync_copy(medge.at[pl.ds(bi * _SUB, _SUB)], dbuf)
        pltpu.sync_copy(dbuf, acc.at[idx_v.at[0]], add=True)
        return carry
    lax.fori_loop(startb, endb, blk_body, 0)
    plsc.subcore_barrier()

    for q in range(_RPT // _ZROWS):
        r0 = sid * _RPT + q * _ZROWS
        pltpu.sync_copy(acc.at[pl.ds(r0, _ZROWS)], out.at[cid, pl.ds(r0, _ZROWS)])


def _edge_block_kernel(hrow_ref, hcol_ref, cd_ref, eattr_ref,
                       w2_ref, b2_ref, wc0_ref, bc0_ref, wc1_ref, m4_ref,
                       medge_ref):
    cd = cd_ref[...]
    radial = jnp.sum(cd * cd, axis=1, keepdims=True)
    ea = eattr_ref[...]
    pre1 = (hrow_ref[...] + hcol_ref[...]
            + jnp.concatenate([ea, radial], axis=1) @ m4_ref[...])
    t1 = _silu(pre1)
    m = _silu(t1 @ w2_ref[...] + b2_ref[...])
    q = _silu(m @ wc0_ref[...] + bc0_ref[...])
    s = jnp.sum(q * wc1_ref[...], axis=1, keepdims=True)
    eb = m.shape[0]
    medge_ref[...] = jnp.concatenate(
        [m, cd * s, jnp.ones((eb, 1), jnp.float32),
         jnp.zeros((eb, _ROWW - 68), jnp.float32)], axis=1)


def _run_edge_block(hrow, hcol, cd, eattr, w2, b2, wc0, bc0, wc1, m4):
    n_edges = hrow.shape[0]
    grid = n_edges // EDGE_BLOCK
    eb = EDGE_BLOCK
    bs_e = lambda w: pl.BlockSpec((eb, w), lambda i: (i, 0))
    bs_c = lambda a, b: pl.BlockSpec((a, b), lambda i: (0, 0))
    medge = pl.pallas_call(
        _edge_block_kernel,
        grid=(grid,),
        in_specs=[bs_e(64), bs_e(64), bs_e(3), bs_e(4),
                  bs_c(64, 64), bs_c(1, 64), bs_c(64, 64), bs_c(1, 64),
                  bs_c(1, 64), bs_c(5, 64)],
        out_specs=[bs_e(_ROWW)],
        out_shape=[jax.ShapeDtypeStruct((n_edges, _ROWW), jnp.float32)],
    )(hrow, hcol, cd, eattr, w2, b2, wc0, bc0, wc1, m4)
    return medge[0]


def kernel(t, context, x, pos, eigvecs, edge_attr, params, edge_index, batch_ids):
    with jax.default_matmul_precision("float32"):
        return _forward_impl(t, context, x, pos, eigvecs, edge_attr, params,
                             edge_index, batch_ids)


def _forward_impl(t, context, x, pos, eigvecs, edge_attr, params, edge_index, batch_ids):
    f32 = jnp.float32
    pe = jnp.where(jnp.isnan(eigvecs), 0.0, eigvecs) @ params["pe_enc"]["W"] + params["pe_enc"]["b"]
    tg = _timestep_embedding(t, TIME_DIM_C)
    onehot_n = (batch_ids[:, None] == jnp.arange(N_GRAPHS_C)[None, :]).astype(f32)
    time_emb = onehot_n @ tg
    ctx = onehot_n @ (context @ params["context_emb"]["W"] + params["context_emb"]["b"])
    h_node = x @ params["node_emb"]["W"] + params["node_emb"]["b"]
    h = jnp.concatenate([h_node, pe, time_emb, ctx], axis=1)

    row = edge_index[0]
    col = edge_index[1]
    n = h.shape[0]
    row2 = row.reshape(_NBLK, _SUB)

    oh16 = (batch_ids[:N_GRAPHS_C, None] == jnp.arange(N_GRAPHS_C)[None, :]).astype(f32)
    ttab = oh16 @ tg

    we = params["edge_emb"]["W"]
    be = params["edge_emb"]["b"]

    conv = params["convs"][0]
    h = h @ conv["emb_in"]["W"] + conv["emb_in"]["b"]
    p = pos

    for gcl in conv["gcls"]:
        w1 = gcl["edge_mlp"][0]["W"]
        b1 = gcl["edge_mlp"][0]["b"]
        w1a, w1b = w1[0:64], w1[64:128]
        wr = w1[128:129]
        w1e = w1[129:177]
        w1t = w1[177:193]
        m4 = jnp.concatenate([we @ w1e, wr], axis=0)
        tvec = ttab @ w1t + (be @ w1e + b1)[None, :]
        hA = h @ w1a + onehot_n @ tvec
        hB = h @ w1b

        hrow = hA[row]
        hcol = hB[col]
        cd = p[row] - p[col]

        w2, b2 = gcl["edge_mlp"][1]["W"], gcl["edge_mlp"][1]["b"]
        wc0, bc0 = gcl["coord_mlp"][0]["W"], gcl["coord_mlp"][0]["b"]
        wc1 = gcl["coord_mlp"][1]["W"].T
        medge = _run_edge_block(hrow, hcol, cd, edge_attr, w2,
                                b2[None, :], wc0, bc0[None, :], wc1, m4)

        parts = _sc_scatter(medge, row2)
        tot = parts[0, :n] + parts[1, :n]
        agg = tot[:, :64]
        trans_sum = tot[:, 64:67]
        cnt = tot[:, 67:68]
        p = p + trans_sum / jnp.maximum(cnt, 1.0)

        wn0, bn0 = gcl["node_mlp"][0]["W"], gcl["node_mlp"][0]["b"]
        wn1, bn1 = gcl["node_mlp"][1]["W"], gcl["node_mlp"][1]["b"]
        hid = _silu(h @ wn0[:64] + agg @ wn0[64:] + bn0)
        h = h + (hid @ wn1 + bn1)

    h = h @ conv["emb_out"]["W"] + conv["emb_out"]["b"]

    hg = onehot_n.T @ h
    mlp = params["mlp"]
    out = jax.nn.relu(hg @ mlp[0]["W"] + mlp[0]["b"])
    out = jax.nn.relu(out @ mlp[1]["W"] + mlp[1]["b"])
    out = out @ mlp[2]["W"] + mlp[2]["b"]
    return out


# speedup vs baseline: 1.9142x; 1.9142x over previous
import math
import functools

import jax
import jax.numpy as jnp
from jax import lax
from jax.experimental import pallas as pl
from jax.experimental.pallas import tpu as pltpu
from jax.experimental.pallas import tpu_sc as plsc

N_NODES_C = 10000
N_EDGES_C = 160000
N_GRAPHS_C = 16
TIME_DIM_C = 16

EDGE_BLOCK = 3200

_NC, _NS = 2, 16
_NW = _NC * _NS
_SUB = 128
_NBLK = N_EDGES_C // _SUB
_NPAD = 10240
_RPT = _NPAD // _NS
_ZROWS = 128
_ROWW = 80

_sc_mesh = plsc.VectorSubcoreMesh(core_axis_name="c", subcore_axis_name="s")


def _silu(x):
    return x * jax.nn.sigmoid(x)


def _timestep_embedding(timesteps, dim, max_period=10000):
    half = dim // 2
    freqs = jnp.exp(-math.log(max_period) * jnp.arange(0, half, dtype=jnp.float32) / half)
    args = timesteps[:, None].astype(jnp.float32) * freqs[None]
    return jnp.concatenate([jnp.cos(args), jnp.sin(args)], axis=-1)


@functools.partial(
    pl.kernel,
    out_type=jax.ShapeDtypeStruct((_NC, _NPAD, _ROWW), jnp.float32),
    mesh=_sc_mesh,
    scratch_types=[
        pltpu.VMEM((1, _SUB), jnp.int32),
        pltpu.VMEM((_SUB, _ROWW), jnp.float32),
        pltpu.VMEM((_ZROWS, _ROWW), jnp.float32),
        pltpu.VMEM_SHARED((_NPAD, _ROWW), jnp.float32),
    ],
    compiler_params=pltpu.CompilerParams(use_tc_tiling_on_sc=False),
)
def _sc_scatter(medge, row2, out, idx_v, dbuf, zbuf, acc):
    cid = lax.axis_index("c")
    sid = lax.axis_index("s")
    wid = sid * _NC + cid

    def zrow(r, carry):
        for q in range(_ROWW // 16):
            zbuf[r, pl.ds(q * 16, 16)] = jnp.zeros((16,), jnp.float32)
        return carry
    lax.fori_loop(0, _ZROWS, zrow, 0)
    for q in range(_RPT // _ZROWS):
        pltpu.sync_copy(zbuf, acc.at[pl.ds(sid * _RPT + q * _ZROWS, _ZROWS)])
    plsc.subcore_barrier()

    startb = wid * (_NBLK // _NW) + jnp.minimum(wid, _NBLK % _NW)
    endb = startb + (_NBLK // _NW) + jnp.where(wid < _NBLK % _NW, 1, 0)

    def blk_body(bi, carry):
        pltpu.sync_copy(row2.at[pl.ds(bi, 1)], idx_v)
        pltpu.sync_copy(medge.at[pl.ds(bi * _SUB, _SUB)], dbuf)
        pltpu.sync_copy(dbuf, acc.at[idx_v.at[0]], add=True)
        return carry
    lax.fori_loop(startb, endb, blk_body, 0)
    plsc.subcore_barrier()

    for q in range(_RPT // _ZROWS):
        r0 = sid * _RPT + q * _ZROWS
        pltpu.sync_copy(acc.at[pl.ds(r0, _ZROWS)], out.at[cid, pl.ds(r0, _ZROWS)])


@functools.partial(
    pl.kernel,
    out_type=[jax.ShapeDtypeStruct((N_EDGES_C, _ROWW), jnp.float32),
              jax.ShapeDtypeStruct((N_EDGES_C, _ROWW), jnp.float32)],
    mesh=_sc_mesh,
    scratch_types=[
        pltpu.VMEM((1, _SUB), jnp.int32),
        pltpu.VMEM((1, _SUB), jnp.int32),
        pltpu.VMEM((_SUB, _ROWW), jnp.float32),
        pltpu.VMEM((_SUB, _ROWW), jnp.float32),
        pltpu.SemaphoreType.DMA,
        pltpu.SemaphoreType.DMA,
    ],
    compiler_params=pltpu.CompilerParams(use_tc_tiling_on_sc=False),
)
def _sc_gather(tab_a, tab_b, row2, col2, out_a, out_b,
               idxr, idxc, buf_a, buf_b, sem_a, sem_b):
    cid = lax.axis_index("c")
    sid = lax.axis_index("s")
    wid = sid * _NC + cid
    startb = wid * (_NBLK // _NW) + jnp.minimum(wid, _NBLK % _NW)
    endb = startb + (_NBLK // _NW) + jnp.where(wid < _NBLK % _NW, 1, 0)

    def blk_body(bi, carry):
        pltpu.sync_copy(row2.at[pl.ds(bi, 1)], idxr)
        pltpu.sync_copy(col2.at[pl.ds(bi, 1)], idxc)
        cp_a = pltpu.async_copy(tab_a.at[idxr.at[0]], buf_a, sem_a)
        cp_b = pltpu.async_copy(tab_b.at[idxc.at[0]], buf_b, sem_b)
        cp_a.wait()
        cp_b.wait()
        pltpu.sync_copy(buf_a, out_a.at[pl.ds(bi * _SUB, _SUB)])
        pltpu.sync_copy(buf_b, out_b.at[pl.ds(bi * _SUB, _SUB)])
        return carry
    lax.fori_loop(startb, endb, blk_body, 0)


def _edge_block_kernel(ga_ref, gb_ref, eattr_ref,
                       w2_ref, b2_ref, wc0_ref, bc0_ref, wc1_ref, m4_ref,
                       medge_ref):
    ga = ga_ref[...]
    gb = gb_ref[...]
    cd = ga[:, 64:67] - gb[:, 64:67]
    radial = jnp.sum(cd * cd, axis=1, keepdims=True)
    ea = eattr_ref[...]
    pre1 = (ga[:, :64] + gb[:, :64]
            + jnp.concatenate([ea, radial], axis=1) @ m4_ref[...])
    t1 = _silu(pre1)
    m = _silu(t1 @ w2_ref[...] + b2_ref[...])
    q = _silu(m @ wc0_ref[...] + bc0_ref[...])
    s = jnp.sum(q * wc1_ref[...], axis=1, keepdims=True)
    eb = m.shape[0]
    medge_ref[...] = jnp.concatenate(
        [m, cd * s, jnp.ones((eb, 1), jnp.float32),
         jnp.zeros((eb, _ROWW - 68), jnp.float32)], axis=1)


def _run_edge_block(ga, gb, eattr, w2, b2, wc0, bc0, wc1, m4):
    n_edges = ga.shape[0]
    grid = n_edges // EDGE_BLOCK
    eb = EDGE_BLOCK
    bs_e = lambda w: pl.BlockSpec((eb, w), lambda i: (i, 0))
    bs_c = lambda a, b: pl.BlockSpec((a, b), lambda i: (0, 0))
    medge = pl.pallas_call(
        _edge_block_kernel,
        grid=(grid,),
        in_specs=[bs_e(_ROWW), bs_e(_ROWW), bs_e(4),
                  bs_c(64, 64), bs_c(1, 64), bs_c(64, 64), bs_c(1, 64),
                  bs_c(1, 64), bs_c(5, 64)],
        out_specs=[bs_e(_ROWW)],
        out_shape=[jax.ShapeDtypeStruct((n_edges, _ROWW), jnp.float32)],
    )(ga, gb, eattr, w2, b2, wc0, bc0, wc1, m4)
    return medge[0]


def kernel(t, context, x, pos, eigvecs, edge_attr, params, edge_index, batch_ids):
    with jax.default_matmul_precision("float32"):
        return _forward_impl(t, context, x, pos, eigvecs, edge_attr, params,
                             edge_index, batch_ids)


def _forward_impl(t, context, x, pos, eigvecs, edge_attr, params, edge_index, batch_ids):
    f32 = jnp.float32
    pe = jnp.where(jnp.isnan(eigvecs), 0.0, eigvecs) @ params["pe_enc"]["W"] + params["pe_enc"]["b"]
    tg = _timestep_embedding(t, TIME_DIM_C)
    onehot_n = (batch_ids[:, None] == jnp.arange(N_GRAPHS_C)[None, :]).astype(f32)
    time_emb = onehot_n @ tg
    ctx = onehot_n @ (context @ params["context_emb"]["W"] + params["context_emb"]["b"])
    h_node = x @ params["node_emb"]["W"] + params["node_emb"]["b"]
    h = jnp.concatenate([h_node, pe, time_emb, ctx], axis=1)

    row = edge_index[0]
    col = edge_index[1]
    n = h.shape[0]
    row2 = row.reshape(_NBLK, _SUB)
    col2 = col.reshape(_NBLK, _SUB)
    zpad = jnp.zeros((n, _ROWW - 67), f32)

    oh16 = (batch_ids[:N_GRAPHS_C, None] == jnp.arange(N_GRAPHS_C)[None, :]).astype(f32)
    ttab = oh16 @ tg

    we = params["edge_emb"]["W"]
    be = params["edge_emb"]["b"]

    conv = params["convs"][0]
    h = h @ conv["emb_in"]["W"] + conv["emb_in"]["b"]
    p = pos

    for gcl in conv["gcls"]:
        w1 = gcl["edge_mlp"][0]["W"]
        b1 = gcl["edge_mlp"][0]["b"]
        w1a, w1b = w1[0:64], w1[64:128]
        wr = w1[128:129]
        w1e = w1[129:177]
        w1t = w1[177:193]
        m4 = jnp.concatenate([we @ w1e, wr], axis=0)
        tvec = ttab @ w1t + (be @ w1e + b1)[None, :]
        hA = h @ w1a + onehot_n @ tvec
        hB = h @ w1b
        tab_a = jnp.concatenate([hA, p, zpad], axis=1)
        tab_b = jnp.concatenate([hB, p, zpad], axis=1)

        ga, gb = _sc_gather(tab_a, tab_b, row2, col2)

        w2, b2 = gcl["edge_mlp"][1]["W"], gcl["edge_mlp"][1]["b"]
        wc0, bc0 = gcl["coord_mlp"][0]["W"], gcl["coord_mlp"][0]["b"]
        wc1 = gcl["coord_mlp"][1]["W"].T
        medge = _run_edge_block(ga, gb, edge_attr, w2,
                                b2[None, :], wc0, bc0[None, :], wc1, m4)

        parts = _sc_scatter(medge, row2)
        tot = parts[0, :n] + parts[1, :n]
        agg = tot[:, :64]
        trans_sum = tot[:, 64:67]
        cnt = tot[:, 67:68]
        p = p + trans_sum / jnp.maximum(cnt, 1.0)

        wn0, bn0 = gcl["node_mlp"][0]["W"], gcl["node_mlp"][0]["b"]
        wn1, bn1 = gcl["node_mlp"][1]["W"], gcl["node_mlp"][1]["b"]
        hid = _silu(h @ wn0[:64] + agg @ wn0[64:] + bn0)
        h = h + (hid @ wn1 + bn1)

    h = h @ conv["emb_out"]["W"] + conv["emb_out"]["b"]

    hg = onehot_n.T @ h
    mlp = params["mlp"]
    out = jax.nn.relu(hg @ mlp[0]["W"] + mlp[0]["b"])
    out = jax.nn.relu(out @ mlp[1]["W"] + mlp[1]["b"])
    out = out @ mlp[2]["W"] + mlp[2]["b"]
    return out


# speedup vs baseline: 3.0045x; 1.5695x over previous
import math
import functools

import jax
import jax.numpy as jnp
from jax import lax
from jax.experimental import pallas as pl
from jax.experimental.pallas import tpu as pltpu
from jax.experimental.pallas import tpu_sc as plsc

N_NODES_C = 10000
N_EDGES_C = 160000
N_GRAPHS_C = 16
TIME_DIM_C = 16

EDGE_BLOCK = 3200

_NC, _NS = 2, 16
_NW = _NC * _NS
_SUB = 128
_NBLK = N_EDGES_C // _SUB
_NPAD = 10240
_RPT = _NPAD // _NS
_ZROWS = 128
_ROWW = 80

_sc_mesh = plsc.VectorSubcoreMesh(core_axis_name="c", subcore_axis_name="s")


def _silu(x):
    return x * jax.nn.sigmoid(x)


def _timestep_embedding(timesteps, dim, max_period=10000):
    half = dim // 2
    freqs = jnp.exp(-math.log(max_period) * jnp.arange(0, half, dtype=jnp.float32) / half)
    args = timesteps[:, None].astype(jnp.float32) * freqs[None]
    return jnp.concatenate([jnp.cos(args), jnp.sin(args)], axis=-1)


@functools.partial(
    pl.kernel,
    out_type=jax.ShapeDtypeStruct((_NC, _NPAD, _ROWW), jnp.float32),
    mesh=_sc_mesh,
    scratch_types=[
        pltpu.VMEM((1, _SUB), jnp.int32),
        pltpu.VMEM((_SUB, _ROWW), jnp.float32),
        pltpu.VMEM((_ZROWS, _ROWW), jnp.float32),
        pltpu.VMEM_SHARED((_NPAD, _ROWW), jnp.float32),
    ],
    compiler_params=pltpu.CompilerParams(use_tc_tiling_on_sc=False),
)
def _sc_scatter(medge, row2, out, idx_v, dbuf, zbuf, acc):
    cid = lax.axis_index("c")
    sid = lax.axis_index("s")
    wid = sid * _NC + cid

    def zrow(r, carry):
        for q in range(_ROWW // 16):
            zbuf[r, pl.ds(q * 16, 16)] = jnp.zeros((16,), jnp.float32)
        return carry
    lax.fori_loop(0, _ZROWS, zrow, 0)
    for q in range(_RPT // _ZROWS):
        pltpu.sync_copy(zbuf, acc.at[pl.ds(sid * _RPT + q * _ZROWS, _ZROWS)])
    plsc.subcore_barrier()

    startb = wid * (_NBLK // _NW) + jnp.minimum(wid, _NBLK % _NW)
    endb = startb + (_NBLK // _NW) + jnp.where(wid < _NBLK % _NW, 1, 0)

    def blk_body(bi, carry):
        pltpu.sync_copy(row2.at[pl.ds(bi, 1)], idx_v)
        pltpu.sync_copy(medge.at[pl.ds(bi * _SUB, _SUB)], dbuf)
        pltpu.sync_copy(dbuf, acc.at[idx_v.at[0]], add=True)
        return carry
    lax.fori_loop(startb, endb, blk_body, 0)
    plsc.subcore_barrier()

    for q in range(_RPT // _ZROWS):
        r0 = sid * _RPT + q * _ZROWS
        pltpu.sync_copy(acc.at[pl.ds(r0, _ZROWS)], out.at[cid, pl.ds(r0, _ZROWS)])


@functools.partial(
    pl.kernel,
    out_type=[jax.ShapeDtypeStruct((N_EDGES_C, _ROWW), jnp.float32),
              jax.ShapeDtypeStruct((N_EDGES_C, _ROWW), jnp.float32)],
    mesh=_sc_mesh,
    scratch_types=[
        pltpu.VMEM((1, _SUB), jnp.int32),
        pltpu.VMEM((1, _SUB), jnp.int32),
        pltpu.VMEM((_SUB, _ROWW), jnp.float32),
        pltpu.VMEM((_SUB, _ROWW), jnp.float32),
        pltpu.SemaphoreType.DMA,
        pltpu.SemaphoreType.DMA,
    ],
    compiler_params=pltpu.CompilerParams(use_tc_tiling_on_sc=False),
)
def _sc_gather(tab_a, tab_b, row2, col2, out_a, out_b,
               idxr, idxc, buf_a, buf_b, sem_a, sem_b):
    cid = lax.axis_index("c")
    sid = lax.axis_index("s")
    wid = sid * _NC + cid
    startb = wid * (_NBLK // _NW) + jnp.minimum(wid, _NBLK % _NW)
    endb = startb + (_NBLK // _NW) + jnp.where(wid < _NBLK % _NW, 1, 0)

    def blk_body(bi, carry):
        pltpu.sync_copy(row2.at[pl.ds(bi, 1)], idxr)
        pltpu.sync_copy(col2.at[pl.ds(bi, 1)], idxc)
        cp_a = pltpu.async_copy(tab_a.at[idxr.at[0]], buf_a, sem_a)
        cp_b = pltpu.async_copy(tab_b.at[idxc.at[0]], buf_b, sem_b)
        cp_a.wait()
        cp_b.wait()
        pltpu.sync_copy(buf_a, out_a.at[pl.ds(bi * _SUB, _SUB)])
        pltpu.sync_copy(buf_b, out_b.at[pl.ds(bi * _SUB, _SUB)])
        return carry
    lax.fori_loop(startb, endb, blk_body, 0)


def _edge_block_kernel(ga_ref, gb_ref, eattr_ref,
                       w2_ref, b2_ref, wc0_ref, bc0_ref, wc1_ref, m4_ref,
                       medge_ref):
    ga = ga_ref[...]
    gb = gb_ref[...]
    cd = ga[:, 64:67] - gb[:, 64:67]
    radial = jnp.sum(cd * cd, axis=1, keepdims=True)
    ea = eattr_ref[...]
    with jax.default_matmul_precision("default"):
        pre1 = (ga[:, :64] + gb[:, :64]
                + jnp.concatenate([ea, radial], axis=1) @ m4_ref[...])
        t1 = _silu(pre1)
        m = _silu(t1 @ w2_ref[...] + b2_ref[...])
        q = _silu(m @ wc0_ref[...] + bc0_ref[...])
    s = jnp.sum(q * wc1_ref[...], axis=1, keepdims=True)
    eb = m.shape[0]
    medge_ref[...] = jnp.concatenate(
        [m, cd * s, jnp.ones((eb, 1), jnp.float32),
         jnp.zeros((eb, _ROWW - 68), jnp.float32)], axis=1)


def _run_edge_block(ga, gb, eattr, w2, b2, wc0, bc0, wc1, m4):
    n_edges = ga.shape[0]
    grid = n_edges // EDGE_BLOCK
    eb = EDGE_BLOCK
    bs_e = lambda w: pl.BlockSpec((eb, w), lambda i: (i, 0))
    bs_c = lambda a, b: pl.BlockSpec((a, b), lambda i: (0, 0))
    medge = pl.pallas_call(
        _edge_block_kernel,
        grid=(grid,),
        in_specs=[bs_e(_ROWW), bs_e(_ROWW), bs_e(4),
                  bs_c(64, 64), bs_c(1, 64), bs_c(64, 64), bs_c(1, 64),
                  bs_c(1, 64), bs_c(5, 64)],
        out_specs=[bs_e(_ROWW)],
        out_shape=[jax.ShapeDtypeStruct((n_edges, _ROWW), jnp.float32)],
    )(ga, gb, eattr, w2, b2, wc0, bc0, wc1, m4)
    return medge[0]


def kernel(t, context, x, pos, eigvecs, edge_attr, params, edge_index, batch_ids):
    with jax.default_matmul_precision("float32"):
        return _forward_impl(t, context, x, pos, eigvecs, edge_attr, params,
                             edge_index, batch_ids)


def _forward_impl(t, context, x, pos, eigvecs, edge_attr, params, edge_index, batch_ids):
    f32 = jnp.float32
    pe = jnp.where(jnp.isnan(eigvecs), 0.0, eigvecs) @ params["pe_enc"]["W"] + params["pe_enc"]["b"]
    tg = _timestep_embedding(t, TIME_DIM_C)
    onehot_n = (batch_ids[:, None] == jnp.arange(N_GRAPHS_C)[None, :]).astype(f32)
    time_emb = onehot_n @ tg
    ctx = onehot_n @ (context @ params["context_emb"]["W"] + params["context_emb"]["b"])
    h_node = x @ params["node_emb"]["W"] + params["node_emb"]["b"]
    h = jnp.concatenate([h_node, pe, time_emb, ctx], axis=1)

    row = edge_index[0]
    col = edge_index[1]
    n = h.shape[0]
    row2 = row.reshape(_NBLK, _SUB)
    col2 = col.reshape(_NBLK, _SUB)
    zpad = jnp.zeros((n, _ROWW - 67), f32)

    oh16 = (batch_ids[:N_GRAPHS_C, None] == jnp.arange(N_GRAPHS_C)[None, :]).astype(f32)
    ttab = oh16 @ tg

    we = params["edge_emb"]["W"]
    be = params["edge_emb"]["b"]

    conv = params["convs"][0]
    h = h @ conv["emb_in"]["W"] + conv["emb_in"]["b"]
    p = pos

    for gcl in conv["gcls"]:
        w1 = gcl["edge_mlp"][0]["W"]
        b1 = gcl["edge_mlp"][0]["b"]
        w1a, w1b = w1[0:64], w1[64:128]
        wr = w1[128:129]
        w1e = w1[129:177]
        w1t = w1[177:193]
        m4 = jnp.concatenate([we @ w1e, wr], axis=0)
        tvec = ttab @ w1t + (be @ w1e + b1)[None, :]
        hA = h @ w1a + onehot_n @ tvec
        hB = h @ w1b
        tab_a = jnp.concatenate([hA, p, zpad], axis=1)
        tab_b = jnp.concatenate([hB, p, zpad], axis=1)

        ga, gb = _sc_gather(tab_a, tab_b, row2, col2)

        w2, b2 = gcl["edge_mlp"][1]["W"], gcl["edge_mlp"][1]["b"]
        wc0, bc0 = gcl["coord_mlp"][0]["W"], gcl["coord_mlp"][0]["b"]
        wc1 = gcl["coord_mlp"][1]["W"].T
        medge = _run_edge_block(ga, gb, edge_attr, w2,
                                b2[None, :], wc0, bc0[None, :], wc1, m4)

        parts = _sc_scatter(medge, row2)
        tot = parts[0, :n] + parts[1, :n]
        agg = tot[:, :64]
        trans_sum = tot[:, 64:67]
        cnt = tot[:, 67:68]
        p = p + trans_sum / jnp.maximum(cnt, 1.0)

        wn0, bn0 = gcl["node_mlp"][0]["W"], gcl["node_mlp"][0]["b"]
        wn1, bn1 = gcl["node_mlp"][1]["W"], gcl["node_mlp"][1]["b"]
        hid = _silu(h @ wn0[:64] + agg @ wn0[64:] + bn0)
        h = h + (hid @ wn1 + bn1)

    h = h @ conv["emb_out"]["W"] + conv["emb_out"]["b"]

    hg = onehot_n.T @ h
    mlp = params["mlp"]
    out = jax.nn.relu(hg @ mlp[0]["W"] + mlp[0]["b"])
    out = jax.nn.relu(out @ mlp[1]["W"] + mlp[1]["b"])
    out = out @ mlp[2]["W"] + mlp[2]["b"]
    return out


# speedup vs baseline: 3.4627x; 1.1525x over previous
import math
import functools

import jax
import jax.numpy as jnp
from jax import lax
from jax.experimental import pallas as pl
from jax.experimental.pallas import tpu as pltpu
from jax.experimental.pallas import tpu_sc as plsc

N_NODES_C = 10000
N_EDGES_C = 160000
N_GRAPHS_C = 16
TIME_DIM_C = 16

EDGE_BLOCK = 3200

_NC, _NS = 2, 16
_NW = _NC * _NS
_SUB = 128
_NBLK = N_EDGES_C // _SUB
_NPAD = 10240
_RPT = _NPAD // _NS
_ZROWS = 128
_ROWW = 80
_BPW = _NBLK // _NW + 1
_NBLK_PAD = _NW * _BPW

_sc_mesh = plsc.VectorSubcoreMesh(core_axis_name="c", subcore_axis_name="s")


def _silu(x):
    return x * jax.nn.sigmoid(x)


def _timestep_embedding(timesteps, dim, max_period=10000):
    half = dim // 2
    freqs = jnp.exp(-math.log(max_period) * jnp.arange(0, half, dtype=jnp.float32) / half)
    args = timesteps[:, None].astype(jnp.float32) * freqs[None]
    return jnp.concatenate([jnp.cos(args), jnp.sin(args)], axis=-1)


@functools.partial(
    pl.kernel,
    out_type=jax.ShapeDtypeStruct((_NC, _NPAD, _ROWW), jnp.float32),
    mesh=_sc_mesh,
    scratch_types=[
        pltpu.VMEM((_BPW, _SUB), jnp.int32),
        pltpu.VMEM((_SUB, _ROWW), jnp.float32),
        pltpu.VMEM((_SUB, _ROWW), jnp.float32),
        pltpu.VMEM((_ZROWS, _ROWW), jnp.float32),
        pltpu.VMEM_SHARED((_NPAD, _ROWW), jnp.float32),
        pltpu.SemaphoreType.DMA,
        pltpu.SemaphoreType.DMA,
    ],
    compiler_params=pltpu.CompilerParams(use_tc_tiling_on_sc=False),
)
def _sc_scatter(medge, row2, out, idx_v, dbuf0, dbuf1, zbuf, acc, sem0, sem1):
    cid = lax.axis_index("c")
    sid = lax.axis_index("s")
    wid = sid * _NC + cid

    def zrow(r, carry):
        for q in range(_ROWW // 16):
            zbuf[r, pl.ds(q * 16, 16)] = jnp.zeros((16,), jnp.float32)
        return carry
    lax.fori_loop(0, _ZROWS, zrow, 0)
    for q in range(_RPT // _ZROWS):
        pltpu.sync_copy(zbuf, acc.at[pl.ds(sid * _RPT + q * _ZROWS, _ZROWS)])
    plsc.subcore_barrier()

    startb = wid * (_NBLK // _NW) + jnp.minimum(wid, _NBLK % _NW)
    nb = (_NBLK // _NW) + jnp.where(wid < _NBLK % _NW, 1, 0)

    pltpu.sync_copy(row2.at[pl.ds(startb, _BPW)], idx_v)

    def fire(j, buf, sem):
        jl = jnp.minimum(j, nb - 1)
        pltpu.async_copy(medge.at[pl.ds((startb + jl) * _SUB, _SUB)], buf, sem)

    def drain_scatter(j, buf, sem):
        pltpu.make_async_copy(medge.at[pl.ds(0, _SUB)], buf, sem).wait()

        @pl.when(j < nb)
        def _():
            pltpu.sync_copy(buf, acc.at[idx_v.at[j]], add=True)

    fire(0, dbuf0, sem0)

    def pair(k, carry):
        fire(2 * k + 1, dbuf1, sem1)
        drain_scatter(2 * k, dbuf0, sem0)

        @pl.when(k < _BPW // 2 - 1)
        def _():
            fire(2 * k + 2, dbuf0, sem0)
        drain_scatter(2 * k + 1, dbuf1, sem1)
        return carry
    lax.fori_loop(0, _BPW // 2, pair, 0)
    plsc.subcore_barrier()

    for q in range(_RPT // _ZROWS):
        r0 = sid * _RPT + q * _ZROWS
        pltpu.sync_copy(acc.at[pl.ds(r0, _ZROWS)], out.at[cid, pl.ds(r0, _ZROWS)])


@functools.partial(
    pl.kernel,
    out_type=[jax.ShapeDtypeStruct((N_EDGES_C, _ROWW), jnp.float32),
              jax.ShapeDtypeStruct((N_EDGES_C, _ROWW), jnp.float32)],
    mesh=_sc_mesh,
    scratch_types=[
        pltpu.VMEM((_BPW, _SUB), jnp.int32),
        pltpu.VMEM((_BPW, _SUB), jnp.int32),
        pltpu.VMEM((_SUB, _ROWW), jnp.float32),
        pltpu.VMEM((_SUB, _ROWW), jnp.float32),
        pltpu.VMEM((_SUB, _ROWW), jnp.float32),
        pltpu.VMEM((_SUB, _ROWW), jnp.float32),
        pltpu.SemaphoreType.DMA,
        pltpu.SemaphoreType.DMA,
        pltpu.SemaphoreType.DMA,
        pltpu.SemaphoreType.DMA,
    ],
    compiler_params=pltpu.CompilerParams(use_tc_tiling_on_sc=False),
)
def _sc_gather(tab_a, tab_b, row2, col2, out_a, out_b,
               idxr, idxc, buf_a0, buf_b0, buf_a1, buf_b1,
               sem_a0, sem_b0, sem_a1, sem_b1):
    cid = lax.axis_index("c")
    sid = lax.axis_index("s")
    wid = sid * _NC + cid
    startb = wid * (_NBLK // _NW) + jnp.minimum(wid, _NBLK % _NW)
    nb = (_NBLK // _NW) + jnp.where(wid < _NBLK % _NW, 1, 0)

    pltpu.sync_copy(row2.at[pl.ds(startb, _BPW)], idxr)
    pltpu.sync_copy(col2.at[pl.ds(startb, _BPW)], idxc)

    def fire(j, ba, bb, sa, sb):
        jl = jnp.minimum(j, nb - 1)
        pltpu.async_copy(tab_a.at[idxr.at[jl]], ba, sa)
        pltpu.async_copy(tab_b.at[idxc.at[jl]], bb, sb)

    def drain_write(j, ba, bb, sa, sb):
        jl = jnp.minimum(j, nb - 1)
        pltpu.make_async_copy(tab_a.at[idxr.at[0]], ba, sa).wait()
        pltpu.make_async_copy(tab_b.at[idxc.at[0]], bb, sb).wait()
        off = (startb + jl) * _SUB
        pltpu.sync_copy(ba, out_a.at[pl.ds(off, _SUB)])
        pltpu.sync_copy(bb, out_b.at[pl.ds(off, _SUB)])

    fire(0, buf_a0, buf_b0, sem_a0, sem_b0)

    def pair(k, carry):
        fire(2 * k + 1, buf_a1, buf_b1, sem_a1, sem_b1)
        drain_write(2 * k, buf_a0, buf_b0, sem_a0, sem_b0)

        @pl.when(k < _BPW // 2 - 1)
        def _():
            fire(2 * k + 2, buf_a0, buf_b0, sem_a0, sem_b0)
        drain_write(2 * k + 1, buf_a1, buf_b1, sem_a1, sem_b1)
        return carry
    lax.fori_loop(0, _BPW // 2, pair, 0)


def _edge_block_kernel(ga_ref, gb_ref, eattr_ref,
                       w2_ref, b2_ref, wc0_ref, bc0_ref, wc1_ref, m4_ref,
                       medge_ref):
    ga = ga_ref[...]
    gb = gb_ref[...]
    cd = ga[:, 64:67] - gb[:, 64:67]
    radial = jnp.sum(cd * cd, axis=1, keepdims=True)
    ea = eattr_ref[...]
    with jax.default_matmul_precision("default"):
        pre1 = (ga[:, :64] + gb[:, :64]
                + jnp.concatenate([ea, radial], axis=1) @ m4_ref[...])
        t1 = _silu(pre1)
        m = _silu(t1 @ w2_ref[...] + b2_ref[...])
        q = _silu(m @ wc0_ref[...] + bc0_ref[...])
    s = jnp.sum(q * wc1_ref[...], axis=1, keepdims=True)
    eb = m.shape[0]
    medge_ref[...] = jnp.concatenate(
        [m, cd * s, jnp.ones((eb, 1), jnp.float32),
         jnp.zeros((eb, _ROWW - 68), jnp.float32)], axis=1)


def _run_edge_block(ga, gb, eattr, w2, b2, wc0, bc0, wc1, m4):
    n_edges = ga.shape[0]
    grid = n_edges // EDGE_BLOCK
    eb = EDGE_BLOCK
    bs_e = lambda w: pl.BlockSpec((eb, w), lambda i: (i, 0))
    bs_c = lambda a, b: pl.BlockSpec((a, b), lambda i: (0, 0))
    medge = pl.pallas_call(
        _edge_block_kernel,
        grid=(grid,),
        in_specs=[bs_e(_ROWW), bs_e(_ROWW), bs_e(4),
                  bs_c(64, 64), bs_c(1, 64), bs_c(64, 64), bs_c(1, 64),
                  bs_c(1, 64), bs_c(5, 64)],
        out_specs=[bs_e(_ROWW)],
        out_shape=[jax.ShapeDtypeStruct((n_edges, _ROWW), jnp.float32)],
    )(ga, gb, eattr, w2, b2, wc0, bc0, wc1, m4)
    return medge[0]


def kernel(t, context, x, pos, eigvecs, edge_attr, params, edge_index, batch_ids):
    with jax.default_matmul_precision("float32"):
        return _forward_impl(t, context, x, pos, eigvecs, edge_attr, params,
                             edge_index, batch_ids)


def _forward_impl(t, context, x, pos, eigvecs, edge_attr, params, edge_index, batch_ids):
    f32 = jnp.float32
    pe = jnp.where(jnp.isnan(eigvecs), 0.0, eigvecs) @ params["pe_enc"]["W"] + params["pe_enc"]["b"]
    tg = _timestep_embedding(t, TIME_DIM_C)
    onehot_n = (batch_ids[:, None] == jnp.arange(N_GRAPHS_C)[None, :]).astype(f32)
    time_emb = onehot_n @ tg
    ctx = onehot_n @ (context @ params["context_emb"]["W"] + params["context_emb"]["b"])
    h_node = x @ params["node_emb"]["W"] + params["node_emb"]["b"]
    h = jnp.concatenate([h_node, pe, time_emb, ctx], axis=1)

    row = edge_index[0]
    col = edge_index[1]
    n = h.shape[0]
    ipad = jnp.zeros(((_NBLK_PAD - _NBLK) * _SUB,), row.dtype)
    row2 = jnp.concatenate([row, ipad]).reshape(_NBLK_PAD, _SUB)
    col2 = jnp.concatenate([col, ipad]).reshape(_NBLK_PAD, _SUB)
    zpad = jnp.zeros((n, _ROWW - 67), f32)

    oh16 = (batch_ids[:N_GRAPHS_C, None] == jnp.arange(N_GRAPHS_C)[None, :]).astype(f32)
    ttab = oh16 @ tg

    we = params["edge_emb"]["W"]
    be = params["edge_emb"]["b"]

    conv = params["convs"][0]
    h = h @ conv["emb_in"]["W"] + conv["emb_in"]["b"]
    p = pos

    for gcl in conv["gcls"]:
        w1 = gcl["edge_mlp"][0]["W"]
        b1 = gcl["edge_mlp"][0]["b"]
        w1a, w1b = w1[0:64], w1[64:128]
        wr = w1[128:129]
        w1e = w1[129:177]
        w1t = w1[177:193]
        m4 = jnp.concatenate([we @ w1e, wr], axis=0)
        tvec = ttab @ w1t + (be @ w1e + b1)[None, :]
        hA = h @ w1a + onehot_n @ tvec
        hB = h @ w1b
        tab_a = jnp.concatenate([hA, p, zpad], axis=1)
        tab_b = jnp.concatenate([hB, p, zpad], axis=1)

        ga, gb = _sc_gather(tab_a, tab_b, row2, col2)

        w2, b2 = gcl["edge_mlp"][1]["W"], gcl["edge_mlp"][1]["b"]
        wc0, bc0 = gcl["coord_mlp"][0]["W"], gcl["coord_mlp"][0]["b"]
        wc1 = gcl["coord_mlp"][1]["W"].T
        medge = _run_edge_block(ga, gb, edge_attr, w2,
                                b2[None, :], wc0, bc0[None, :], wc1, m4)

        parts = _sc_scatter(medge, row2)
        tot = parts[0, :n] + parts[1, :n]
        agg = tot[:, :64]
        trans_sum = tot[:, 64:67]
        cnt = tot[:, 67:68]
        p = p + trans_sum / jnp.maximum(cnt, 1.0)

        wn0, bn0 = gcl["node_mlp"][0]["W"], gcl["node_mlp"][0]["b"]
        wn1, bn1 = gcl["node_mlp"][1]["W"], gcl["node_mlp"][1]["b"]
        hid = _silu(h @ wn0[:64] + agg @ wn0[64:] + bn0)
        h = h + (hid @ wn1 + bn1)

    h = h @ conv["emb_out"]["W"] + conv["emb_out"]["b"]

    hg = onehot_n.T @ h
    mlp = params["mlp"]
    out = jax.nn.relu(hg @ mlp[0]["W"] + mlp[0]["b"])
    out = jax.nn.relu(out @ mlp[1]["W"] + mlp[1]["b"])
    out = out @ mlp[2]["W"] + mlp[2]["b"]
    return out


# speedup vs baseline: 3.9848x; 1.1508x over previous
import math
import functools

import jax
import jax.numpy as jnp
from jax import lax
from jax.experimental import pallas as pl
from jax.experimental.pallas import tpu as pltpu
from jax.experimental.pallas import tpu_sc as plsc

N_NODES_C = 10000
N_EDGES_C = 160000
N_GRAPHS_C = 16
TIME_DIM_C = 16

EDGE_BLOCK = 8000

_NC, _NS = 2, 16
_NW = _NC * _NS
_SUB = 128
_NBLK = N_EDGES_C // _SUB
_NPAD = 10240
_RPT = _NPAD // _NS
_ZROWS = 128
_ROWW = 72
_GATW = 40
_BPW = _NBLK // _NW + 1
_NBLK_PAD = _NW * _BPW

_sc_mesh = plsc.VectorSubcoreMesh(core_axis_name="c", subcore_axis_name="s")


def _silu(x):
    return x * jax.nn.sigmoid(x)


def _timestep_embedding(timesteps, dim, max_period=10000):
    half = dim // 2
    freqs = jnp.exp(-math.log(max_period) * jnp.arange(0, half, dtype=jnp.float32) / half)
    args = timesteps[:, None].astype(jnp.float32) * freqs[None]
    return jnp.concatenate([jnp.cos(args), jnp.sin(args)], axis=-1)


@functools.partial(
    pl.kernel,
    out_type=jax.ShapeDtypeStruct((_NC, _NPAD, _ROWW), jnp.float32),
    mesh=_sc_mesh,
    scratch_types=[
        pltpu.VMEM((_BPW, _SUB), jnp.int32),
        pltpu.VMEM((_SUB, _ROWW), jnp.float32),
        pltpu.VMEM((_SUB, _ROWW), jnp.float32),
        pltpu.VMEM((_ZROWS, _ROWW), jnp.float32),
        pltpu.VMEM_SHARED((_NPAD, _ROWW), jnp.float32),
        pltpu.SemaphoreType.DMA,
        pltpu.SemaphoreType.DMA,
    ],
    compiler_params=pltpu.CompilerParams(use_tc_tiling_on_sc=False),
)
def _sc_scatter(medge, row2, out, idx_v, dbuf0, dbuf1, zbuf, acc, sem0, sem1):
    cid = lax.axis_index("c")
    sid = lax.axis_index("s")
    wid = sid * _NC + cid

    def zrow(r, carry):
        for q in range(_ROWW // 16):
            zbuf[r, pl.ds(q * 16, 16)] = jnp.zeros((16,), jnp.float32)
        return carry
    lax.fori_loop(0, _ZROWS, zrow, 0)
    for q in range(_RPT // _ZROWS):
        pltpu.sync_copy(zbuf, acc.at[pl.ds(sid * _RPT + q * _ZROWS, _ZROWS)])
    plsc.subcore_barrier()

    startb = wid * (_NBLK // _NW) + jnp.minimum(wid, _NBLK % _NW)
    nb = (_NBLK // _NW) + jnp.where(wid < _NBLK % _NW, 1, 0)

    pltpu.sync_copy(row2.at[pl.ds(startb, _BPW)], idx_v)

    def fire(j, buf, sem):
        jl = jnp.minimum(j, nb - 1)
        pltpu.async_copy(medge.at[pl.ds((startb + jl) * _SUB, _SUB)], buf, sem)

    def drain_scatter(j, buf, sem):
        pltpu.make_async_copy(medge.at[pl.ds(0, _SUB)], buf, sem).wait()

        @pl.when(j < nb)
        def _():
            pltpu.sync_copy(buf, acc.at[idx_v.at[j]], add=True)

    fire(0, dbuf0, sem0)

    def pair(k, carry):
        fire(2 * k + 1, dbuf1, sem1)
        drain_scatter(2 * k, dbuf0, sem0)

        @pl.when(k < _BPW // 2 - 1)
        def _():
            fire(2 * k + 2, dbuf0, sem0)
        drain_scatter(2 * k + 1, dbuf1, sem1)
        return carry
    lax.fori_loop(0, _BPW // 2, pair, 0)
    plsc.subcore_barrier()

    for q in range(_RPT // _ZROWS):
        r0 = sid * _RPT + q * _ZROWS
        pltpu.sync_copy(acc.at[pl.ds(r0, _ZROWS)], out.at[cid, pl.ds(r0, _ZROWS)])


@functools.partial(
    pl.kernel,
    out_type=[jax.ShapeDtypeStruct((N_EDGES_C, _GATW), jnp.float32),
              jax.ShapeDtypeStruct((N_EDGES_C, _GATW), jnp.float32)],
    mesh=_sc_mesh,
    scratch_types=[
        pltpu.VMEM((_BPW, _SUB), jnp.int32),
        pltpu.VMEM((_BPW, _SUB), jnp.int32),
        pltpu.VMEM((_SUB, _GATW), jnp.float32),
        pltpu.VMEM((_SUB, _GATW), jnp.float32),
        pltpu.VMEM((_SUB, _GATW), jnp.float32),
        pltpu.VMEM((_SUB, _GATW), jnp.float32),
        pltpu.SemaphoreType.DMA,
        pltpu.SemaphoreType.DMA,
        pltpu.SemaphoreType.DMA,
        pltpu.SemaphoreType.DMA,
    ],
    compiler_params=pltpu.CompilerParams(use_tc_tiling_on_sc=False),
)
def _sc_gather(tab_a, tab_b, row2, col2, out_a, out_b,
               idxr, idxc, buf_a0, buf_b0, buf_a1, buf_b1,
               sem_a0, sem_b0, sem_a1, sem_b1):
    cid = lax.axis_index("c")
    sid = lax.axis_index("s")
    wid = sid * _NC + cid
    startb = wid * (_NBLK // _NW) + jnp.minimum(wid, _NBLK % _NW)
    nb = (_NBLK // _NW) + jnp.where(wid < _NBLK % _NW, 1, 0)

    pltpu.sync_copy(row2.at[pl.ds(startb, _BPW)], idxr)
    pltpu.sync_copy(col2.at[pl.ds(startb, _BPW)], idxc)

    def fire(j, ba, bb, sa, sb):
        jl = jnp.minimum(j, nb - 1)
        pltpu.async_copy(tab_a.at[idxr.at[jl]], ba, sa)
        pltpu.async_copy(tab_b.at[idxc.at[jl]], bb, sb)

    def drain_write(j, ba, bb, sa, sb):
        jl = jnp.minimum(j, nb - 1)
        pltpu.make_async_copy(tab_a.at[idxr.at[0]], ba, sa).wait()
        pltpu.make_async_copy(tab_b.at[idxc.at[0]], bb, sb).wait()
        off = (startb + jl) * _SUB
        pltpu.sync_copy(ba, out_a.at[pl.ds(off, _SUB)])
        pltpu.sync_copy(bb, out_b.at[pl.ds(off, _SUB)])

    fire(0, buf_a0, buf_b0, sem_a0, sem_b0)

    def pair(k, carry):
        fire(2 * k + 1, buf_a1, buf_b1, sem_a1, sem_b1)
        drain_write(2 * k, buf_a0, buf_b0, sem_a0, sem_b0)

        @pl.when(k < _BPW // 2 - 1)
        def _():
            fire(2 * k + 2, buf_a0, buf_b0, sem_a0, sem_b0)
        drain_write(2 * k + 1, buf_a1, buf_b1, sem_a1, sem_b1)
        return carry
    lax.fori_loop(0, _BPW // 2, pair, 0)


def _edge_block_kernel(ga_ref, gb_ref, eattr_ref,
                       w2_ref, b2_ref, wc0_ref, bc0_ref, wc1_ref, m4_ref,
                       medge_ref):
    ga = ga_ref[...]
    gb = gb_ref[...]
    eb = ga.shape[0]

    def _unpack(g):
        wi = jax.lax.bitcast_convert_type(g[:, :32], jnp.int32)
        hi = jax.lax.bitcast_convert_type(wi & jnp.int32(-65536), jnp.float32)
        lo = jax.lax.bitcast_convert_type(wi << 16, jnp.float32)
        return hi, lo

    ha_hi, ha_lo = _unpack(ga)
    hb_hi, hb_lo = _unpack(gb)
    hsum = jnp.concatenate([ha_hi + hb_hi, ha_lo + hb_lo], axis=1)
    cd = ga[:, 32:35] - gb[:, 32:35]
    radial = jnp.sum(cd * cd, axis=1, keepdims=True)
    ea = eattr_ref[...]
    with jax.default_matmul_precision("default"):
        pre1 = hsum + jnp.concatenate([ea, radial], axis=1) @ m4_ref[...]
        t1 = _silu(pre1)
        m = _silu(t1 @ w2_ref[...] + b2_ref[...])
        q = _silu(m @ wc0_ref[...] + bc0_ref[...])
    s = jnp.sum(q * wc1_ref[...], axis=1, keepdims=True)
    medge_ref[...] = jnp.concatenate(
        [m, cd * s, jnp.ones((eb, 1), jnp.float32),
         jnp.zeros((eb, _ROWW - 68), jnp.float32)], axis=1)


def _run_edge_block(ga, gb, eattr, w2, b2, wc0, bc0, wc1, m4):
    n_edges = ga.shape[0]
    grid = n_edges // EDGE_BLOCK
    eb = EDGE_BLOCK
    bs_e = lambda w: pl.BlockSpec((eb, w), lambda i: (i, 0))
    bs_c = lambda a, b: pl.BlockSpec((a, b), lambda i: (0, 0))
    medge = pl.pallas_call(
        _edge_block_kernel,
        grid=(grid,),
        in_specs=[bs_e(_GATW), bs_e(_GATW), bs_e(4),
                  bs_c(64, 64), bs_c(1, 64), bs_c(64, 64), bs_c(1, 64),
                  bs_c(1, 64), bs_c(5, 64)],
        out_specs=[bs_e(_ROWW)],
        out_shape=[jax.ShapeDtypeStruct((n_edges, _ROWW), jnp.float32)],
    )(ga, gb, eattr, w2, b2, wc0, bc0, wc1, m4)
    return medge[0]


def kernel(t, context, x, pos, eigvecs, edge_attr, params, edge_index, batch_ids):
    with jax.default_matmul_precision("float32"):
        return _forward_impl(t, context, x, pos, eigvecs, edge_attr, params,
                             edge_index, batch_ids)


def _forward_impl(t, context, x, pos, eigvecs, edge_attr, params, edge_index, batch_ids):
    f32 = jnp.float32
    pe = jnp.where(jnp.isnan(eigvecs), 0.0, eigvecs) @ params["pe_enc"]["W"] + params["pe_enc"]["b"]
    tg = _timestep_embedding(t, TIME_DIM_C)
    onehot_n = (batch_ids[:, None] == jnp.arange(N_GRAPHS_C)[None, :]).astype(f32)
    time_emb = onehot_n @ tg
    ctx = onehot_n @ (context @ params["context_emb"]["W"] + params["context_emb"]["b"])
    h_node = x @ params["node_emb"]["W"] + params["node_emb"]["b"]
    h = jnp.concatenate([h_node, pe, time_emb, ctx], axis=1)

    row = edge_index[0]
    col = edge_index[1]
    n = h.shape[0]
    ipad = jnp.zeros(((_NBLK_PAD - _NBLK) * _SUB,), row.dtype)
    row2 = jnp.concatenate([row, ipad]).reshape(_NBLK_PAD, _SUB)
    col2 = jnp.concatenate([col, ipad]).reshape(_NBLK_PAD, _SUB)
    zpad = jnp.zeros((n, _GATW - 35), f32)

    def _pack_tab(hmat, pp):
        hbits = jax.lax.bitcast_convert_type(hmat.astype(jnp.bfloat16), jnp.uint16)
        words = (hbits[:, :32].astype(jnp.uint32) << 16) | hbits[:, 32:].astype(jnp.uint32)
        hw = jax.lax.bitcast_convert_type(words, jnp.float32)
        return jnp.concatenate([hw, pp, zpad], axis=1)

    oh16 = (batch_ids[:N_GRAPHS_C, None] == jnp.arange(N_GRAPHS_C)[None, :]).astype(f32)
    ttab = oh16 @ tg

    we = params["edge_emb"]["W"]
    be = params["edge_emb"]["b"]

    conv = params["convs"][0]
    h = h @ conv["emb_in"]["W"] + conv["emb_in"]["b"]
    p = pos

    for gcl in conv["gcls"]:
        w1 = gcl["edge_mlp"][0]["W"]
        b1 = gcl["edge_mlp"][0]["b"]
        w1a, w1b = w1[0:64], w1[64:128]
        wr = w1[128:129]
        w1e = w1[129:177]
        w1t = w1[177:193]
        m4 = jnp.concatenate([we @ w1e, wr], axis=0)
        tvec = ttab @ w1t + (be @ w1e + b1)[None, :]
        hA = h @ w1a + onehot_n @ tvec
        hB = h @ w1b
        tab_a = _pack_tab(hA, p)
        tab_b = _pack_tab(hB, p)

        ga, gb = _sc_gather(tab_a, tab_b, row2, col2)

        w2, b2 = gcl["edge_mlp"][1]["W"], gcl["edge_mlp"][1]["b"]
        wc0, bc0 = gcl["coord_mlp"][0]["W"], gcl["coord_mlp"][0]["b"]
        wc1 = gcl["coord_mlp"][1]["W"].T
        medge = _run_edge_block(ga, gb, edge_attr, w2,
                                b2[None, :], wc0, bc0[None, :], wc1, m4)

        parts = _sc_scatter(medge, row2)
        tot = parts[0, :n] + parts[1, :n]
        agg = tot[:, :64]
        trans_sum = tot[:, 64:67]
        cnt = tot[:, 67:68]
        p = p + trans_sum / jnp.maximum(cnt, 1.0)

        wn0, bn0 = gcl["node_mlp"][0]["W"], gcl["node_mlp"][0]["b"]
        wn1, bn1 = gcl["node_mlp"][1]["W"], gcl["node_mlp"][1]["b"]
        hid = _silu(h @ wn0[:64] + agg @ wn0[64:] + bn0)
        h = h + (hid @ wn1 + bn1)

    h = h @ conv["emb_out"]["W"] + conv["emb_out"]["b"]

    hg = onehot_n.T @ h
    mlp = params["mlp"]
    out = jax.nn.relu(hg @ mlp[0]["W"] + mlp[0]["b"])
    out = jax.nn.relu(out @ mlp[1]["W"] + mlp[1]["b"])
    out = out @ mlp[2]["W"] + mlp[2]["b"]
    return out
